# baseline (speedup 1.0000x reference)
import functools

import jax
import jax.numpy as jnp
from jax import lax
from jax.experimental import pallas as pl


_F32 = jnp.float32


def _dot(a, b):
    return lax.dot_general(a, b, (((1,), (0,)), ((), ())),
                           preferred_element_type=_F32)


def _relu(x):
    return jnp.maximum(x, 0.0)


def _setup_kernel(objs_ref, boxes_ref, shapes_ref, teb_ref, tes_ref,
                  wb_ref, bb_ref, ws_ref, bs_ref, ovb_ref, ovs_ref):
    n = objs_ref.shape[0]
    nobj = teb_ref.shape[0]
    onehot = (lax.broadcasted_iota(jnp.int32, (n, nobj), 1)
              == objs_ref[...]).astype(_F32)
    emb_b = _dot(onehot, teb_ref[...])
    emb_s = _dot(onehot, tes_ref[...])
    bx = _dot(boxes_ref[...], wb_ref[...]) + bb_ref[...]
    sh = _dot(shapes_ref[...], ws_ref[...]) + bs_ref[...]
    ovb_ref[...] = jnp.concatenate([emb_b, bx], axis=1)
    ovs_ref[...] = jnp.concatenate([emb_s, sh], axis=1)


def _node_setup(objs, boxes_gt, shapes_gt, params):
    n = objs.shape[0]
    emb = params['obj_emb_box'].shape[1]
    (wb, bb), = params['box_emb']
    (ws, bs), = params['shape_emb']
    out_sh = jax.ShapeDtypeStruct((n, 2 * emb), _F32)
    return pl.pallas_call(
        _setup_kernel,
        out_shape=(out_sh, out_sh),
    )(objs.reshape(n, 1).astype(jnp.int32), boxes_gt, shapes_gt,
      params['obj_emb_box'], params['obj_emb_shape'],
      wb, bb.reshape(1, -1), ws, bs.reshape(1, -1))


def _pred_kernel(p_ref, tb_ref, ts_ref, pvb_ref, pvs_ref):
    eblk = p_ref.shape[0]
    npred = tb_ref.shape[0]
    onehot = (lax.broadcasted_iota(jnp.int32, (eblk, npred), 1)
              == p_ref[...]).astype(_F32)
    pvb_ref[...] = _dot(onehot, tb_ref[...])
    pvs_ref[...] = _dot(onehot, ts_ref[...])


def _pred_setup(p, params, eblk):
    e = p.shape[0]
    tb = params['pred_emb_box']
    ts = params['pred_emb_shape']
    d = tb.shape[1]
    grid = e // eblk
    out_sh = jax.ShapeDtypeStruct((e, d), _F32)
    return pl.pallas_call(
        _pred_kernel,
        grid=(grid,),
        in_specs=[
            pl.BlockSpec((eblk, 1), lambda i: (i, 0)),
            pl.BlockSpec(tb.shape, lambda i: (0, 0)),
            pl.BlockSpec(ts.shape, lambda i: (0, 0)),
        ],
        out_specs=(pl.BlockSpec((eblk, d), lambda i: (i, 0)),
                   pl.BlockSpec((eblk, d), lambda i: (i, 0))),
        out_shape=(out_sh, out_sh),
    )(p.reshape(e, 1).astype(jnp.int32), tb, ts)


def _counts_kernel(srow_ref, orow_ref, cnt_ref):
    n = cnt_ref.shape[0]
    eblk = srow_ref.shape[-1]

    @pl.when(pl.program_id(0) == 0)
    def _():
        cnt_ref[...] = jnp.zeros_like(cnt_ref)

    ii = lax.broadcasted_iota(jnp.int32, (n, eblk), 0)
    ohs = (ii == srow_ref[0]).astype(_F32)
    oho = (ii == orow_ref[0]).astype(_F32)
    cnt_ref[...] += (jnp.sum(ohs, axis=1, keepdims=True)
                     + jnp.sum(oho, axis=1, keepdims=True))


def _edge_counts(s_row3, o_row3, n, eblk):
    grid = s_row3.shape[0]
    return pl.pallas_call(
        _counts_kernel,
        grid=(grid,),
        in_specs=[
            pl.BlockSpec((1, 1, eblk), lambda i: (i, 0, 0)),
            pl.BlockSpec((1, 1, eblk), lambda i: (i, 0, 0)),
        ],
        out_specs=pl.BlockSpec((n, 1), lambda i: (0, 0)),
        out_shape=jax.ShapeDtypeStruct((n, 1), _F32),
    )(s_row3, o_row3)


_BF16 = jnp.bfloat16


def _hilo(x):
    hi = x.astype(_BF16)
    lo = (x - hi.astype(_F32)).astype(_BF16)
    return hi, lo


def _premul_kernel(ov_ref, w1a_ref, ash_ref, asl_ref, aoh_ref, aol_ref):
    din = ov_ref.shape[1]
    w = w1a_ref[...]
    a_s = _dot(ov_ref[...], w[:din, :])
    a_o = _dot(ov_ref[...], w[2 * din:, :])
    ash_ref[...], asl_ref[...] = _hilo(a_s)
    aoh_ref[...], aol_ref[...] = _hilo(a_o)


def _edge_kernel(scol_ref, ocol_ref, srow_ref, orow_ref, pred_ref,
                 ash_ref, asl_ref, aoh_ref, aol_ref,
                 w1p_ref, b1a_ref, w1b_ref, b1b_ref,
                 newp_ref, pooled_ref, *, hid, din):
    eblk = scol_ref.shape[0]
    n = ash_ref.shape[0]

    ii_g = lax.broadcasted_iota(jnp.int32, (eblk, n), 1)
    oh_gs = (ii_g == scol_ref[...]).astype(_BF16)
    oh_go = (ii_g == ocol_ref[...]).astype(_BF16)
    gs = _dot(oh_gs, ash_ref[...]) + _dot(oh_gs, asl_ref[...])
    go = _dot(oh_go, aoh_ref[...]) + _dot(oh_go, aol_ref[...])
    q = _dot(pred_ref[...], w1p_ref[...])
    t1 = _relu(gs + go + q + b1a_ref[...])
    u = _relu(_dot(t1, w1b_ref[...]) + b1b_ref[...])

    newp_ref[...] = u[:, hid:hid + din]

    ii_s = lax.broadcasted_iota(jnp.int32, (n, eblk), 0)
    oh_ss = (ii_s == srow_ref[0]).astype(_BF16)
    oh_so = (ii_s == orow_ref[0]).astype(_BF16)

    @pl.when(pl.program_id(0) == 0)
    def _():
        pooled_ref[...] = jnp.zeros_like(pooled_ref)

    ns_hi, ns_lo = _hilo(u[:, :hid])
    no_hi, no_lo = _hilo(u[:, hid + din:])
    pooled_ref[...] += ((_dot(oh_ss, ns_hi) + _dot(oh_ss, ns_lo))
                        + (_dot(oh_so, no_hi) + _dot(oh_so, no_lo)))


def _node_kernel(pooled_ref, cnt_ref, w2a_ref, b2a_ref, w2b_ref, b2b_ref,
                 out_ref):
    pm = pooled_ref[...] / jnp.clip(cnt_ref[...], 1.0, None)
    h = _relu(_dot(pm, w2a_ref[...]) + b2a_ref[...])
    out_ref[...] = _relu(_dot(h, w2b_ref[...]) + b2b_ref[...])


def _gtc_layer(obj_vecs, pred_vecs, idx, counts, layer, eblk):
    n, din = obj_vecs.shape
    e = pred_vecs.shape[0]
    s_col, o_col, s_row3, o_row3 = idx
    (w1a, b1a), (w1b, b1b) = layer['net1']
    (w2a, b2a), (w2b, b2b) = layer['net2']
    hid = w1a.shape[1]
    grid = e // eblk

    bf_sh = jax.ShapeDtypeStruct((n, hid), _BF16)
    a_sh, a_sl, a_oh, a_ol = pl.pallas_call(
        _premul_kernel,
        out_shape=(bf_sh, bf_sh, bf_sh, bf_sh),
    )(obj_vecs, w1a)

    w1p = w1a[din:2 * din, :]

    new_p, pooled = pl.pallas_call(
        functools.partial(_edge_kernel, hid=hid, din=din),
        grid=(grid,),
        in_specs=[
            pl.BlockSpec((eblk, 1), lambda i: (i, 0)),
            pl.BlockSpec((eblk, 1), lambda i: (i, 0)),
            pl.BlockSpec((1, 1, eblk), lambda i: (i, 0, 0)),
            pl.BlockSpec((1, 1, eblk), lambda i: (i, 0, 0)),
            pl.BlockSpec((eblk, din), lambda i: (i, 0)),
            pl.BlockSpec((n, hid), lambda i: (0, 0)),
            pl.BlockSpec((n, hid), lambda i: (0, 0)),
            pl.BlockSpec((n, hid), lambda i: (0, 0)),
            pl.BlockSpec((n, hid), lambda i: (0, 0)),
            pl.BlockSpec((din, hid), lambda i: (0, 0)),
            pl.BlockSpec((1, hid), lambda i: (0, 0)),
            pl.BlockSpec((hid, 2 * hid + din), lambda i: (0, 0)),
            pl.BlockSpec((1, 2 * hid + din), lambda i: (0, 0)),
        ],
        out_specs=(pl.BlockSpec((eblk, din), lambda i: (i, 0)),
                   pl.BlockSpec((n, hid), lambda i: (0, 0))),
        out_shape=(jax.ShapeDtypeStruct((e, din), _F32),
                   jax.ShapeDtypeStruct((n, hid), _F32)),
    )(s_col, o_col, s_row3, o_row3, pred_vecs, a_sh, a_sl, a_oh, a_ol, w1p,
      b1a.reshape(1, -1), w1b, b1b.reshape(1, -1))

    new_obj = pl.pallas_call(
        _node_kernel,
        out_shape=jax.ShapeDtypeStruct((n, din), _F32),
    )(pooled, counts, w2a, b2a.reshape(1, -1), w2b, b2b.reshape(1, -1))

    return new_obj, new_p


def _heads_kernel(ovb_ref, ovs_ref,
                  wbh0_ref, bbh0_ref, wbh1_ref, bbh1_ref,
                  wbm_ref, bbm_ref, wbv_ref, bbv_ref,
                  wsh0_ref, bsh0_ref, wsh1_ref, bsh1_ref,
                  wsm_ref, bsm_ref, wsv_ref, bsv_ref,
                  mub_ref, lvb_ref, mus_ref, lvs_ref):
    hb = _relu(_dot(ovb_ref[...], wbh0_ref[...]) + bbh0_ref[...])
    hb = _relu(_dot(hb, wbh1_ref[...]) + bbh1_ref[...])
    mub_ref[...] = _dot(hb, wbm_ref[...]) + bbm_ref[...]
    lvb_ref[...] = _dot(hb, wbv_ref[...]) + bbv_ref[...]
    hs = _relu(_dot(ovs_ref[...], wsh0_ref[...]) + bsh0_ref[...])
    hs = _relu(_dot(hs, wsh1_ref[...]) + bsh1_ref[...])
    mus_ref[...] = _dot(hs, wsm_ref[...]) + bsm_ref[...]
    lvs_ref[...] = _dot(hs, wsv_ref[...]) + bsv_ref[...]


def _heads(ovb, ovs, params):
    n = ovb.shape[0]
    (wbh0, bbh0), (wbh1, bbh1) = params['box_mean_var']
    (wbm, bbm), = params['box_mean']
    (wbv, bbv), = params['box_var']
    (wsh0, bsh0), (wsh1, bsh1) = params['shape_mean_var']
    (wsm, bsm), = params['shape_mean']
    (wsv, bsv), = params['shape_var']
    emb = wbm.shape[1]
    out_sh = jax.ShapeDtypeStruct((n, emb), _F32)
    return pl.pallas_call(
        _heads_kernel,
        out_shape=(out_sh, out_sh, out_sh, out_sh),
    )(ovb, ovs,
      wbh0, bbh0.reshape(1, -1), wbh1, bbh1.reshape(1, -1),
      wbm, bbm.reshape(1, -1), wbv, bbv.reshape(1, -1),
      wsh0, bsh0.reshape(1, -1), wsh1, bsh1.reshape(1, -1),
      wsm, bsm.reshape(1, -1), wsv, bsv.reshape(1, -1))


def kernel(objs, triples, boxes_gt, shapes_gt, params):
    e = triples.shape[0]
    n = objs.shape[0]
    eblk = min(512, e)
    grid = e // eblk

    s = triples[:, 0].astype(jnp.int32)
    p = triples[:, 1].astype(jnp.int32)
    o = triples[:, 2].astype(jnp.int32)
    idx = (s.reshape(e, 1), o.reshape(e, 1),
           s.reshape(grid, 1, eblk), o.reshape(grid, 1, eblk))

    ovb, ovs = _node_setup(objs, boxes_gt, shapes_gt, params)
    pvb, pvs = _pred_setup(p, params, eblk)
    counts = _edge_counts(idx[2], idx[3], n, eblk)

    for layer in params['gconv_box']:
        ovb, pvb = _gtc_layer(ovb, pvb, idx, counts, layer, eblk)
    for layer in params['gconv_shape']:
        ovs, pvs = _gtc_layer(ovs, pvs, idx, counts, layer, eblk)

    ov = jnp.concatenate([ovb, ovs], axis=1)
    pv = jnp.concatenate([pvb, pvs], axis=1)
    for layer in params['gconv_shared']:
        ov, pv = _gtc_layer(ov, pv, idx, counts, layer, eblk)

    d = ov.shape[1] // 2
    return _heads(ov[:, :d], ov[:, d:], params)


# speedup vs baseline: 1.9784x; 1.9784x over previous
import functools

import jax
import jax.numpy as jnp
from jax import lax
from jax.experimental import pallas as pl
from jax.experimental.pallas import tpu as pltpu
from jax.experimental.pallas import tpu_sc as plsc


_F32 = jnp.float32
_NC = 2
_NS = 16
_NW = _NC * _NS
_CH = 64


def _dot(a, b):
    return lax.dot_general(a, b, (((1,), (0,)), ((), ())),
                           preferred_element_type=_F32)


def _relu(x):
    return jnp.maximum(x, 0.0)


def _setup_kernel(objs_ref, boxes_ref, shapes_ref, teb_ref, tes_ref,
                  wb_ref, bb_ref, ws_ref, bs_ref, ovb_ref, ovs_ref):
    n = objs_ref.shape[0]
    nobj = teb_ref.shape[0]
    onehot = (lax.broadcasted_iota(jnp.int32, (n, nobj), 1)
              == objs_ref[...]).astype(_F32)
    emb_b = _dot(onehot, teb_ref[...])
    emb_s = _dot(onehot, tes_ref[...])
    bx = _dot(boxes_ref[...], wb_ref[...]) + bb_ref[...]
    sh = _dot(shapes_ref[...], ws_ref[...]) + bs_ref[...]
    ovb_ref[...] = jnp.concatenate([emb_b, bx], axis=1)
    ovs_ref[...] = jnp.concatenate([emb_s, sh], axis=1)


def _node_setup(objs, boxes_gt, shapes_gt, params):
    n = objs.shape[0]
    emb = params['obj_emb_box'].shape[1]
    (wb, bb), = params['box_emb']
    (ws, bs), = params['shape_emb']
    out_sh = jax.ShapeDtypeStruct((n, 2 * emb), _F32)
    return pl.pallas_call(
        _setup_kernel,
        out_shape=(out_sh, out_sh),
    )(objs.reshape(n, 1).astype(jnp.int32), boxes_gt, shapes_gt,
      params['obj_emb_box'], params['obj_emb_shape'],
      wb, bb.reshape(1, -1), ws, bs.reshape(1, -1))


def _pred_kernel(p_ref, tb_ref, ts_ref, pvb_ref, pvs_ref):
    eblk = p_ref.shape[0]
    npred = tb_ref.shape[0]
    onehot = (lax.broadcasted_iota(jnp.int32, (eblk, npred), 1)
              == p_ref[...]).astype(_F32)
    pvb_ref[...] = _dot(onehot, tb_ref[...])
    pvs_ref[...] = _dot(onehot, ts_ref[...])


def _pred_setup(p, params, eblk):
    e = p.shape[0]
    tb = params['pred_emb_box']
    ts = params['pred_emb_shape']
    d = tb.shape[1]
    grid = e // eblk
    out_sh = jax.ShapeDtypeStruct((e, d), _F32)
    return pl.pallas_call(
        _pred_kernel,
        grid=(grid,),
        in_specs=[
            pl.BlockSpec((eblk, 1), lambda i: (i, 0)),
            pl.BlockSpec(tb.shape, lambda i: (0, 0)),
            pl.BlockSpec(ts.shape, lambda i: (0, 0)),
        ],
        out_specs=(pl.BlockSpec((eblk, d), lambda i: (i, 0)),
                   pl.BlockSpec((eblk, d), lambda i: (i, 0))),
        out_shape=(out_sh, out_sh),
    )(p.reshape(e, 1).astype(jnp.int32), tb, ts)


def _counts_kernel(srow_ref, orow_ref, cnt_ref):
    n = cnt_ref.shape[0]
    eblk = srow_ref.shape[-1]

    @pl.when(pl.program_id(0) == 0)
    def _():
        cnt_ref[...] = jnp.zeros_like(cnt_ref)

    ii = lax.broadcasted_iota(jnp.int32, (n, eblk), 0)
    ohs = (ii == srow_ref[0]).astype(_F32)
    oho = (ii == orow_ref[0]).astype(_F32)
    cnt_ref[...] += (jnp.sum(ohs, axis=1, keepdims=True)
                     + jnp.sum(oho, axis=1, keepdims=True))


def _edge_counts(s_row3, o_row3, n, eblk):
    grid = s_row3.shape[0]
    return pl.pallas_call(
        _counts_kernel,
        grid=(grid,),
        in_specs=[
            pl.BlockSpec((1, 1, eblk), lambda i: (i, 0, 0)),
            pl.BlockSpec((1, 1, eblk), lambda i: (i, 0, 0)),
        ],
        out_specs=pl.BlockSpec((n, 1), lambda i: (0, 0)),
        out_shape=jax.ShapeDtypeStruct((n, 1), _F32),
    )(s_row3, o_row3)


def _sc_gather_body(as_hbm, ao_hbm, s3_hbm, o3_hbm, out_hbm,
                    idxs_v, idxo_v, bs, bo, sem, *, per_w, nch, hid):
    wid = lax.axis_index("s") * _NC + lax.axis_index("c")

    def chunk(k, _):
        base = wid * per_w + k * _CH
        pltpu.sync_copy(s3_hbm.at[wid, k], idxs_v)
        pltpu.sync_copy(o3_hbm.at[wid, k], idxo_v)
        pltpu.async_copy(as_hbm.at[idxs_v], bs, sem).wait()
        pltpu.async_copy(ao_hbm.at[idxo_v], bo, sem).wait()

        def row(r, _):
            def col(c, _):
                sl = pl.ds(c * 16, 16)
                bs[r, sl] = bs[r, sl] + bo[r, sl]
                return ()
            return lax.fori_loop(0, hid // 16, col, ())

        lax.fori_loop(0, _CH, row, ())
        pltpu.sync_copy(bs, out_hbm.at[pl.ds(base, _CH)])
        return ()

    lax.fori_loop(0, nch, chunk, ())


def _sc_gather_sum(a_s, a_o, s3, o3):
    n, hid = a_s.shape
    e = s3.shape[0] * s3.shape[1] * s3.shape[2]
    per_w = e // _NW
    nch = per_w // _CH
    mesh = plsc.VectorSubcoreMesh(core_axis_name="c", subcore_axis_name="s")
    body = functools.partial(_sc_gather_body, per_w=per_w, nch=nch, hid=hid)
    return pl.kernel(
        body,
        out_type=jax.ShapeDtypeStruct((e, hid), _F32),
        mesh=mesh,
        scratch_types=[
            pltpu.VMEM((_CH,), jnp.int32),
            pltpu.VMEM((_CH,), jnp.int32),
            pltpu.VMEM((_CH, hid), _F32),
            pltpu.VMEM((_CH, hid), _F32),
            pltpu.SemaphoreType.DMA,
        ],
    )(a_s, a_o, s3, o3)


def _premul_kernel(ov_ref, w1a_ref, as_ref, ao_ref):
    din = ov_ref.shape[1]
    w = w1a_ref[...]
    as_ref[...] = _dot(ov_ref[...], w[:din, :])
    ao_ref[...] = _dot(ov_ref[...], w[2 * din:, :])


def _edge_kernel(srow_ref, orow_ref, gsum_ref, pred_ref,
                 w1p_ref, b1a_ref, w1b_ref, b1b_ref,
                 newp_ref, pooled_ref, *, hid, din):
    n = pooled_ref.shape[0]
    eblk = pred_ref.shape[0]

    q = _dot(pred_ref[...], w1p_ref[...])
    t1 = _relu(gsum_ref[...] + q + b1a_ref[...])
    u = _relu(_dot(t1, w1b_ref[...]) + b1b_ref[...])

    newp_ref[...] = u[:, hid:hid + din]

    ii_s = lax.broadcasted_iota(jnp.int32, (n, eblk), 0)
    oh_ss = (ii_s == srow_ref[0]).astype(_F32)
    oh_so = (ii_s == orow_ref[0]).astype(_F32)

    @pl.when(pl.program_id(0) == 0)
    def _():
        pooled_ref[...] = jnp.zeros_like(pooled_ref)

    pooled_ref[...] += (_dot(oh_ss, u[:, :hid])
                        + _dot(oh_so, u[:, hid + din:]))


def _node_kernel(pooled_ref, cnt_ref, w2a_ref, b2a_ref, w2b_ref, b2b_ref,
                 out_ref):
    pm = pooled_ref[...] / jnp.clip(cnt_ref[...], 1.0, None)
    h = _relu(_dot(pm, w2a_ref[...]) + b2a_ref[...])
    out_ref[...] = _relu(_dot(h, w2b_ref[...]) + b2b_ref[...])


def _gtc_layer(obj_vecs, pred_vecs, idx, counts, layer, eblk):
    n, din = obj_vecs.shape
    e = pred_vecs.shape[0]
    s_row3, o_row3, s3, o3 = idx
    (w1a, b1a), (w1b, b1b) = layer['net1']
    (w2a, b2a), (w2b, b2b) = layer['net2']
    hid = w1a.shape[1]
    grid = e // eblk

    a_s, a_o = pl.pallas_call(
        _premul_kernel,
        out_shape=(jax.ShapeDtypeStruct((n, hid), _F32),
                   jax.ShapeDtypeStruct((n, hid), _F32)),
    )(obj_vecs, w1a)

    gsum = _sc_gather_sum(a_s, a_o, s3, o3)

    w1p = w1a[din:2 * din, :]

    new_p, pooled = pl.pallas_call(
        functools.partial(_edge_kernel, hid=hid, din=din),
        grid=(grid,),
        in_specs=[
            pl.BlockSpec((1, 1, eblk), lambda i: (i, 0, 0)),
            pl.BlockSpec((1, 1, eblk), lambda i: (i, 0, 0)),
            pl.BlockSpec((eblk, hid), lambda i: (i, 0)),
            pl.BlockSpec((eblk, din), lambda i: (i, 0)),
            pl.BlockSpec((din, hid), lambda i: (0, 0)),
            pl.BlockSpec((1, hid), lambda i: (0, 0)),
            pl.BlockSpec((hid, 2 * hid + din), lambda i: (0, 0)),
            pl.BlockSpec((1, 2 * hid + din), lambda i: (0, 0)),
        ],
        out_specs=(pl.BlockSpec((eblk, din), lambda i: (i, 0)),
                   pl.BlockSpec((n, hid), lambda i: (0, 0))),
        out_shape=(jax.ShapeDtypeStruct((e, din), _F32),
                   jax.ShapeDtypeStruct((n, hid), _F32)),
    )(s_row3, o_row3, gsum, pred_vecs, w1p,
      b1a.reshape(1, -1), w1b, b1b.reshape(1, -1))

    new_obj = pl.pallas_call(
        _node_kernel,
        out_shape=jax.ShapeDtypeStruct((n, din), _F32),
    )(pooled, counts, w2a, b2a.reshape(1, -1), w2b, b2b.reshape(1, -1))

    return new_obj, new_p


def _heads_kernel(ovb_ref, ovs_ref,
                  wbh0_ref, bbh0_ref, wbh1_ref, bbh1_ref,
                  wbm_ref, bbm_ref, wbv_ref, bbv_ref,
                  wsh0_ref, bsh0_ref, wsh1_ref, bsh1_ref,
                  wsm_ref, bsm_ref, wsv_ref, bsv_ref,
                  mub_ref, lvb_ref, mus_ref, lvs_ref):
    hb = _relu(_dot(ovb_ref[...], wbh0_ref[...]) + bbh0_ref[...])
    hb = _relu(_dot(hb, wbh1_ref[...]) + bbh1_ref[...])
    mub_ref[...] = _dot(hb, wbm_ref[...]) + bbm_ref[...]
    lvb_ref[...] = _dot(hb, wbv_ref[...]) + bbv_ref[...]
    hs = _relu(_dot(ovs_ref[...], wsh0_ref[...]) + bsh0_ref[...])
    hs = _relu(_dot(hs, wsh1_ref[...]) + bsh1_ref[...])
    mus_ref[...] = _dot(hs, wsm_ref[...]) + bsm_ref[...]
    lvs_ref[...] = _dot(hs, wsv_ref[...]) + bsv_ref[...]


def _heads(ovb, ovs, params):
    n = ovb.shape[0]
    (wbh0, bbh0), (wbh1, bbh1) = params['box_mean_var']
    (wbm, bbm), = params['box_mean']
    (wbv, bbv), = params['box_var']
    (wsh0, bsh0), (wsh1, bsh1) = params['shape_mean_var']
    (wsm, bsm), = params['shape_mean']
    (wsv, bsv), = params['shape_var']
    emb = wbm.shape[1]
    out_sh = jax.ShapeDtypeStruct((n, emb), _F32)
    return pl.pallas_call(
        _heads_kernel,
        out_shape=(out_sh, out_sh, out_sh, out_sh),
    )(ovb, ovs,
      wbh0, bbh0.reshape(1, -1), wbh1, bbh1.reshape(1, -1),
      wbm, bbm.reshape(1, -1), wbv, bbv.reshape(1, -1),
      wsh0, bsh0.reshape(1, -1), wsh1, bsh1.reshape(1, -1),
      wsm, bsm.reshape(1, -1), wsv, bsv.reshape(1, -1))


def kernel(objs, triples, boxes_gt, shapes_gt, params):
    e = triples.shape[0]
    n = objs.shape[0]
    eblk = min(512, e)
    grid = e // eblk

    s = triples[:, 0].astype(jnp.int32)
    p = triples[:, 1].astype(jnp.int32)
    o = triples[:, 2].astype(jnp.int32)
    per_w = e // _NW
    nch = per_w // _CH
    idx = (s.reshape(grid, 1, eblk), o.reshape(grid, 1, eblk),
           s.reshape(_NW, nch, _CH), o.reshape(_NW, nch, _CH))

    ovb, ovs = _node_setup(objs, boxes_gt, shapes_gt, params)
    pvb, pvs = _pred_setup(p, params, eblk)
    counts = _edge_counts(idx[0], idx[1], n, eblk)

    for layer in params['gconv_box']:
        ovb, pvb = _gtc_layer(ovb, pvb, idx, counts, layer, eblk)
    for layer in params['gconv_shape']:
        ovs, pvs = _gtc_layer(ovs, pvs, idx, counts, layer, eblk)

    ov = jnp.concatenate([ovb, ovs], axis=1)
    pv = jnp.concatenate([pvb, pvs], axis=1)
    for layer in params['gconv_shared']:
        ov, pv = _gtc_layer(ov, pv, idx, counts, layer, eblk)

    d = ov.shape[1] // 2
    return _heads(ov[:, :d], ov[:, d:], params)


# speedup vs baseline: 2.4105x; 1.2184x over previous
import functools

import jax
import jax.numpy as jnp
from jax import lax
from jax.experimental import pallas as pl
from jax.experimental.pallas import tpu as pltpu
from jax.experimental.pallas import tpu_sc as plsc


_F32 = jnp.float32
_NC = 2
_NS = 16
_NW = _NC * _NS
_CH = 32


def _dot(a, b):
    return lax.dot_general(a, b, (((1,), (0,)), ((), ())),
                           preferred_element_type=_F32)


def _relu(x):
    return jnp.maximum(x, 0.0)


def _setup_kernel(objs_ref, boxes_ref, shapes_ref, teb_ref, tes_ref,
                  wb_ref, bb_ref, ws_ref, bs_ref, ovb_ref, ovs_ref):
    n = objs_ref.shape[0]
    nobj = teb_ref.shape[0]
    onehot = (lax.broadcasted_iota(jnp.int32, (n, nobj), 1)
              == objs_ref[...]).astype(_F32)
    emb_b = _dot(onehot, teb_ref[...])
    emb_s = _dot(onehot, tes_ref[...])
    bx = _dot(boxes_ref[...], wb_ref[...]) + bb_ref[...]
    sh = _dot(shapes_ref[...], ws_ref[...]) + bs_ref[...]
    ovb_ref[...] = jnp.concatenate([emb_b, bx], axis=1)
    ovs_ref[...] = jnp.concatenate([emb_s, sh], axis=1)


def _node_setup(objs, boxes_gt, shapes_gt, params):
    n = objs.shape[0]
    emb = params['obj_emb_box'].shape[1]
    (wb, bb), = params['box_emb']
    (ws, bs), = params['shape_emb']
    out_sh = jax.ShapeDtypeStruct((n, 2 * emb), _F32)
    return pl.pallas_call(
        _setup_kernel,
        out_shape=(out_sh, out_sh),
    )(objs.reshape(n, 1).astype(jnp.int32), boxes_gt, shapes_gt,
      params['obj_emb_box'], params['obj_emb_shape'],
      wb, bb.reshape(1, -1), ws, bs.reshape(1, -1))


def _pred_kernel(p_ref, tb_ref, ts_ref, pvb_ref, pvs_ref):
    eblk = p_ref.shape[0]
    npred = tb_ref.shape[0]
    onehot = (lax.broadcasted_iota(jnp.int32, (eblk, npred), 1)
              == p_ref[...]).astype(_F32)
    pvb_ref[...] = _dot(onehot, tb_ref[...])
    pvs_ref[...] = _dot(onehot, ts_ref[...])


def _pred_setup(p, params, eblk):
    e = p.shape[0]
    tb = params['pred_emb_box']
    ts = params['pred_emb_shape']
    d = tb.shape[1]
    grid = e // eblk
    out_sh = jax.ShapeDtypeStruct((e, d), _F32)
    return pl.pallas_call(
        _pred_kernel,
        grid=(grid,),
        in_specs=[
            pl.BlockSpec((eblk, 1), lambda i: (i, 0)),
            pl.BlockSpec(tb.shape, lambda i: (0, 0)),
            pl.BlockSpec(ts.shape, lambda i: (0, 0)),
        ],
        out_specs=(pl.BlockSpec((eblk, d), lambda i: (i, 0)),
                   pl.BlockSpec((eblk, d), lambda i: (i, 0))),
        out_shape=(out_sh, out_sh),
    )(p.reshape(e, 1).astype(jnp.int32), tb, ts)


def _counts_kernel(srow_ref, orow_ref, cnt_ref):
    n = cnt_ref.shape[0]
    eblk = srow_ref.shape[-1]

    @pl.when(pl.program_id(0) == 0)
    def _():
        cnt_ref[...] = jnp.zeros_like(cnt_ref)

    ii = lax.broadcasted_iota(jnp.int32, (n, eblk), 0)
    ohs = (ii == srow_ref[0]).astype(_F32)
    oho = (ii == orow_ref[0]).astype(_F32)
    cnt_ref[...] += (jnp.sum(ohs, axis=1, keepdims=True)
                     + jnp.sum(oho, axis=1, keepdims=True))


def _edge_counts(s_row3, o_row3, n, eblk):
    grid = s_row3.shape[0]
    return pl.pallas_call(
        _counts_kernel,
        grid=(grid,),
        in_specs=[
            pl.BlockSpec((1, 1, eblk), lambda i: (i, 0, 0)),
            pl.BlockSpec((1, 1, eblk), lambda i: (i, 0, 0)),
        ],
        out_specs=pl.BlockSpec((n, 1), lambda i: (0, 0)),
        out_shape=jax.ShapeDtypeStruct((n, 1), _F32),
    )(s_row3, o_row3)


def _sc_gather_body(as_hbm, ao_hbm, s3_hbm, o3_hbm, out_hbm,
                    idxs, idxo, bs0, bo0, bs1, bo1, sem0, sem1,
                    *, per_w, nch, hid):
    wid = lax.axis_index("s") * _NC + lax.axis_index("c")
    base_w = wid * per_w
    nv = hid // 16

    pltpu.sync_copy(s3_hbm.at[wid], idxs)
    pltpu.sync_copy(o3_hbm.at[wid], idxo)

    def issue(k, bs, bo, sem):
        @pl.when(k < nch)
        def _():
            pltpu.async_copy(as_hbm.at[idxs.at[k]], bs, sem)
            pltpu.async_copy(ao_hbm.at[idxo.at[k]], bo, sem)

    def drain_add_write(k, bs, bo, sem):
        pltpu.make_async_copy(as_hbm.at[idxs.at[k]], bs, sem).wait()
        pltpu.make_async_copy(ao_hbm.at[idxo.at[k]], bo, sem).wait()

        def row(r, _):
            for c in range(nv):
                sl = pl.ds(c * 16, 16)
                bs[r, sl] = bs[r, sl] + bo[r, sl]
            return ()

        lax.fori_loop(0, _CH, row, ())
        pltpu.sync_copy(bs, out_hbm.at[pl.ds(base_w + k * _CH, _CH)])

    issue(0, bs0, bo0, sem0)

    def pair(i, _):
        k0 = i * 2
        issue(k0 + 1, bs1, bo1, sem1)
        drain_add_write(k0, bs0, bo0, sem0)
        issue(k0 + 2, bs0, bo0, sem0)
        drain_add_write(k0 + 1, bs1, bo1, sem1)
        return ()

    lax.fori_loop(0, nch // 2, pair, ())


def _sc_gather_sum(a_s, a_o, s3, o3):
    n, hid = a_s.shape
    e = s3.shape[0] * s3.shape[1] * s3.shape[2]
    per_w = e // _NW
    nch = per_w // _CH
    mesh = plsc.VectorSubcoreMesh(core_axis_name="c", subcore_axis_name="s")
    body = functools.partial(_sc_gather_body, per_w=per_w, nch=nch, hid=hid)
    buf = pltpu.VMEM((_CH, hid), _F32)
    return pl.kernel(
        body,
        out_type=jax.ShapeDtypeStruct((e, hid), _F32),
        mesh=mesh,
        scratch_types=[
            pltpu.VMEM((nch, _CH), jnp.int32),
            pltpu.VMEM((nch, _CH), jnp.int32),
            buf, buf, buf, buf,
            pltpu.SemaphoreType.DMA,
            pltpu.SemaphoreType.DMA,
        ],
    )(a_s, a_o, s3, o3)


def _premul_kernel(ov_ref, w1a_ref, as_ref, ao_ref):
    din = ov_ref.shape[1]
    w = w1a_ref[...]
    as_ref[...] = _dot(ov_ref[...], w[:din, :])
    ao_ref[...] = _dot(ov_ref[...], w[2 * din:, :])


def _edge_kernel(srow_ref, orow_ref, gsum_ref, pred_ref,
                 w1p_ref, b1a_ref, w1b_ref, b1b_ref,
                 newp_ref, pooled_ref, *, hid, din):
    n = pooled_ref.shape[0]
    eblk = pred_ref.shape[0]

    q = _dot(pred_ref[...], w1p_ref[...])
    t1 = _relu(gsum_ref[...] + q + b1a_ref[...])
    u = _relu(_dot(t1, w1b_ref[...]) + b1b_ref[...])

    newp_ref[...] = u[:, hid:hid + din]

    ii_s = lax.broadcasted_iota(jnp.int32, (n, eblk), 0)
    oh_ss = (ii_s == srow_ref[0]).astype(_F32)
    oh_so = (ii_s == orow_ref[0]).astype(_F32)

    @pl.when(pl.program_id(0) == 0)
    def _():
        pooled_ref[...] = jnp.zeros_like(pooled_ref)

    pooled_ref[...] += (_dot(oh_ss, u[:, :hid])
                        + _dot(oh_so, u[:, hid + din:]))


def _node_kernel(pooled_ref, cnt_ref, w2a_ref, b2a_ref, w2b_ref, b2b_ref,
                 out_ref):
    pm = pooled_ref[...] / jnp.clip(cnt_ref[...], 1.0, None)
    h = _relu(_dot(pm, w2a_ref[...]) + b2a_ref[...])
    out_ref[...] = _relu(_dot(h, w2b_ref[...]) + b2b_ref[...])


def _gtc_layer(obj_vecs, pred_vecs, idx, counts, layer, eblk):
    n, din = obj_vecs.shape
    e = pred_vecs.shape[0]
    s_row3, o_row3, s3, o3 = idx
    (w1a, b1a), (w1b, b1b) = layer['net1']
    (w2a, b2a), (w2b, b2b) = layer['net2']
    hid = w1a.shape[1]
    grid = e // eblk

    a_s, a_o = pl.pallas_call(
        _premul_kernel,
        out_shape=(jax.ShapeDtypeStruct((n, hid), _F32),
                   jax.ShapeDtypeStruct((n, hid), _F32)),
    )(obj_vecs, w1a)

    gsum = _sc_gather_sum(a_s, a_o, s3, o3)

    w1p = w1a[din:2 * din, :]

    new_p, pooled = pl.pallas_call(
        functools.partial(_edge_kernel, hid=hid, din=din),
        grid=(grid,),
        in_specs=[
            pl.BlockSpec((1, 1, eblk), lambda i: (i, 0, 0)),
            pl.BlockSpec((1, 1, eblk), lambda i: (i, 0, 0)),
            pl.BlockSpec((eblk, hid), lambda i: (i, 0)),
            pl.BlockSpec((eblk, din), lambda i: (i, 0)),
            pl.BlockSpec((din, hid), lambda i: (0, 0)),
            pl.BlockSpec((1, hid), lambda i: (0, 0)),
            pl.BlockSpec((hid, 2 * hid + din), lambda i: (0, 0)),
            pl.BlockSpec((1, 2 * hid + din), lambda i: (0, 0)),
        ],
        out_specs=(pl.BlockSpec((eblk, din), lambda i: (i, 0)),
                   pl.BlockSpec((n, hid), lambda i: (0, 0))),
        out_shape=(jax.ShapeDtypeStruct((e, din), _F32),
                   jax.ShapeDtypeStruct((n, hid), _F32)),
    )(s_row3, o_row3, gsum, pred_vecs, w1p,
      b1a.reshape(1, -1), w1b, b1b.reshape(1, -1))

    new_obj = pl.pallas_call(
        _node_kernel,
        out_shape=jax.ShapeDtypeStruct((n, din), _F32),
    )(pooled, counts, w2a, b2a.reshape(1, -1), w2b, b2b.reshape(1, -1))

    return new_obj, new_p


def _heads_kernel(ovb_ref, ovs_ref,
                  wbh0_ref, bbh0_ref, wbh1_ref, bbh1_ref,
                  wbm_ref, bbm_ref, wbv_ref, bbv_ref,
                  wsh0_ref, bsh0_ref, wsh1_ref, bsh1_ref,
                  wsm_ref, bsm_ref, wsv_ref, bsv_ref,
                  mub_ref, lvb_ref, mus_ref, lvs_ref):
    hb = _relu(_dot(ovb_ref[...], wbh0_ref[...]) + bbh0_ref[...])
    hb = _relu(_dot(hb, wbh1_ref[...]) + bbh1_ref[...])
    mub_ref[...] = _dot(hb, wbm_ref[...]) + bbm_ref[...]
    lvb_ref[...] = _dot(hb, wbv_ref[...]) + bbv_ref[...]
    hs = _relu(_dot(ovs_ref[...], wsh0_ref[...]) + bsh0_ref[...])
    hs = _relu(_dot(hs, wsh1_ref[...]) + bsh1_ref[...])
    mus_ref[...] = _dot(hs, wsm_ref[...]) + bsm_ref[...]
    lvs_ref[...] = _dot(hs, wsv_ref[...]) + bsv_ref[...]


def _heads(ovb, ovs, params):
    n = ovb.shape[0]
    (wbh0, bbh0), (wbh1, bbh1) = params['box_mean_var']
    (wbm, bbm), = params['box_mean']
    (wbv, bbv), = params['box_var']
    (wsh0, bsh0), (wsh1, bsh1) = params['shape_mean_var']
    (wsm, bsm), = params['shape_mean']
    (wsv, bsv), = params['shape_var']
    emb = wbm.shape[1]
    out_sh = jax.ShapeDtypeStruct((n, emb), _F32)
    return pl.pallas_call(
        _heads_kernel,
        out_shape=(out_sh, out_sh, out_sh, out_sh),
    )(ovb, ovs,
      wbh0, bbh0.reshape(1, -1), wbh1, bbh1.reshape(1, -1),
      wbm, bbm.reshape(1, -1), wbv, bbv.reshape(1, -1),
      wsh0, bsh0.reshape(1, -1), wsh1, bsh1.reshape(1, -1),
      wsm, bsm.reshape(1, -1), wsv, bsv.reshape(1, -1))


def kernel(objs, triples, boxes_gt, shapes_gt, params):
    e = triples.shape[0]
    n = objs.shape[0]
    eblk = min(512, e)
    grid = e // eblk

    s = triples[:, 0].astype(jnp.int32)
    p = triples[:, 1].astype(jnp.int32)
    o = triples[:, 2].astype(jnp.int32)
    per_w = e // _NW
    nch = per_w // _CH
    idx = (s.reshape(grid, 1, eblk), o.reshape(grid, 1, eblk),
           s.reshape(_NW, nch, _CH), o.reshape(_NW, nch, _CH))

    ovb, ovs = _node_setup(objs, boxes_gt, shapes_gt, params)
    pvb, pvs = _pred_setup(p, params, eblk)
    counts = _edge_counts(idx[0], idx[1], n, eblk)

    for lb, ls in zip(params['gconv_box'], params['gconv_shape']):
        ovb, pvb = _gtc_layer(ovb, pvb, idx, counts, lb, eblk)
        ovs, pvs = _gtc_layer(ovs, pvs, idx, counts, ls, eblk)

    ov = jnp.concatenate([ovb, ovs], axis=1)
    pv = jnp.concatenate([pvb, pvs], axis=1)
    for layer in params['gconv_shared']:
        ov, pv = _gtc_layer(ov, pv, idx, counts, layer, eblk)

    d = ov.shape[1] // 2
    return _heads(ov[:, :d], ov[:, d:], params)


# speedup vs baseline: 2.5683x; 1.0655x over previous
import functools

import jax
import jax.numpy as jnp
from jax import lax
from jax.experimental import pallas as pl
from jax.experimental.pallas import tpu as pltpu
from jax.experimental.pallas import tpu_sc as plsc


_F32 = jnp.float32
_NC = 2
_NS = 16
_NW = _NC * _NS
_CH = 32


def _dot(a, b):
    return lax.dot_general(a, b, (((1,), (0,)), ((), ())),
                           preferred_element_type=_F32)


def _relu(x):
    return jnp.maximum(x, 0.0)


def _setup_kernel(objs_ref, boxes_ref, shapes_ref, teb_ref, tes_ref,
                  wb_ref, bb_ref, ws_ref, bs_ref, ovb_ref, ovs_ref):
    n = objs_ref.shape[0]
    nobj = teb_ref.shape[0]
    onehot = (lax.broadcasted_iota(jnp.int32, (n, nobj), 1)
              == objs_ref[...]).astype(_F32)
    emb_b = _dot(onehot, teb_ref[...])
    emb_s = _dot(onehot, tes_ref[...])
    bx = _dot(boxes_ref[...], wb_ref[...]) + bb_ref[...]
    sh = _dot(shapes_ref[...], ws_ref[...]) + bs_ref[...]
    ovb_ref[...] = jnp.concatenate([emb_b, bx], axis=1)
    ovs_ref[...] = jnp.concatenate([emb_s, sh], axis=1)


def _node_setup(objs, boxes_gt, shapes_gt, params):
    n = objs.shape[0]
    emb = params['obj_emb_box'].shape[1]
    (wb, bb), = params['box_emb']
    (ws, bs), = params['shape_emb']
    out_sh = jax.ShapeDtypeStruct((n, 2 * emb), _F32)
    return pl.pallas_call(
        _setup_kernel,
        out_shape=(out_sh, out_sh),
    )(objs.reshape(n, 1).astype(jnp.int32), boxes_gt, shapes_gt,
      params['obj_emb_box'], params['obj_emb_shape'],
      wb, bb.reshape(1, -1), ws, bs.reshape(1, -1))


def _pred_kernel(p_ref, tb_ref, ts_ref, pvb_ref, pvs_ref):
    eblk = p_ref.shape[0]
    npred = tb_ref.shape[0]
    onehot = (lax.broadcasted_iota(jnp.int32, (eblk, npred), 1)
              == p_ref[...]).astype(_F32)
    pvb_ref[...] = _dot(onehot, tb_ref[...])
    pvs_ref[...] = _dot(onehot, ts_ref[...])


def _pred_setup(p, params, eblk):
    e = p.shape[0]
    tb = params['pred_emb_box']
    ts = params['pred_emb_shape']
    d = tb.shape[1]
    grid = e // eblk
    out_sh = jax.ShapeDtypeStruct((e, d), _F32)
    return pl.pallas_call(
        _pred_kernel,
        grid=(grid,),
        in_specs=[
            pl.BlockSpec((eblk, 1), lambda i: (i, 0)),
            pl.BlockSpec(tb.shape, lambda i: (0, 0)),
            pl.BlockSpec(ts.shape, lambda i: (0, 0)),
        ],
        out_specs=(pl.BlockSpec((eblk, d), lambda i: (i, 0)),
                   pl.BlockSpec((eblk, d), lambda i: (i, 0))),
        out_shape=(out_sh, out_sh),
    )(p.reshape(e, 1).astype(jnp.int32), tb, ts)


def _counts_kernel(srow_ref, orow_ref, cnt_ref):
    n = cnt_ref.shape[0]
    eblk = srow_ref.shape[-1]

    @pl.when(pl.program_id(0) == 0)
    def _():
        cnt_ref[...] = jnp.zeros_like(cnt_ref)

    ii = lax.broadcasted_iota(jnp.int32, (n, eblk), 0)
    ohs = (ii == srow_ref[0]).astype(_F32)
    oho = (ii == orow_ref[0]).astype(_F32)
    cnt_ref[...] += (jnp.sum(ohs, axis=1, keepdims=True)
                     + jnp.sum(oho, axis=1, keepdims=True))


def _edge_counts(s_row3, o_row3, n, eblk):
    grid = s_row3.shape[0]
    return pl.pallas_call(
        _counts_kernel,
        grid=(grid,),
        in_specs=[
            pl.BlockSpec((1, 1, eblk), lambda i: (i, 0, 0)),
            pl.BlockSpec((1, 1, eblk), lambda i: (i, 0, 0)),
        ],
        out_specs=pl.BlockSpec((n, 1), lambda i: (0, 0)),
        out_shape=jax.ShapeDtypeStruct((n, 1), _F32),
    )(s_row3, o_row3)


def _sc_gather_body(as_hbm, ao_hbm, s3_hbm, o3_hbm, out_hbm,
                    idxs, idxo, bs0, bo0, bs1, bo1, sem0, sem1,
                    *, per_w, nch, hid):
    wid = lax.axis_index("s") * _NC + lax.axis_index("c")
    base_w = wid * per_w
    nv = hid // 16

    pltpu.sync_copy(s3_hbm.at[wid], idxs)
    pltpu.sync_copy(o3_hbm.at[wid], idxo)

    def issue(k, bs, bo, sem):
        @pl.when(k < nch)
        def _():
            pltpu.async_copy(as_hbm.at[idxs.at[k]], bs, sem)
            pltpu.async_copy(ao_hbm.at[idxo.at[k]], bo, sem)

    def drain_add_write(k, bs, bo, sem):
        pltpu.make_async_copy(as_hbm.at[idxs.at[k]], bs, sem).wait()
        pltpu.make_async_copy(ao_hbm.at[idxo.at[k]], bo, sem).wait()

        def row(r, _):
            for c in range(nv):
                sl = pl.ds(c * 16, 16)
                bs[r, sl] = bs[r, sl] + bo[r, sl]
            return ()

        lax.fori_loop(0, _CH, row, ())
        pltpu.sync_copy(bs, out_hbm.at[pl.ds(base_w + k * _CH, _CH)])

    issue(0, bs0, bo0, sem0)

    def pair(i, _):
        k0 = i * 2
        issue(k0 + 1, bs1, bo1, sem1)
        drain_add_write(k0, bs0, bo0, sem0)
        issue(k0 + 2, bs0, bo0, sem0)
        drain_add_write(k0 + 1, bs1, bo1, sem1)
        return ()

    lax.fori_loop(0, nch // 2, pair, ())


def _sc_gather_sum(a_s, a_o, s3, o3):
    n, hid = a_s.shape
    e = s3.shape[0] * s3.shape[1] * s3.shape[2]
    per_w = e // _NW
    nch = per_w // _CH
    mesh = plsc.VectorSubcoreMesh(core_axis_name="c", subcore_axis_name="s")
    body = functools.partial(_sc_gather_body, per_w=per_w, nch=nch, hid=hid)
    buf = pltpu.VMEM((_CH, hid), _F32)
    return pl.kernel(
        body,
        out_type=jax.ShapeDtypeStruct((e, hid), _F32),
        mesh=mesh,
        scratch_types=[
            pltpu.VMEM((nch, _CH), jnp.int32),
            pltpu.VMEM((nch, _CH), jnp.int32),
            buf, buf, buf, buf,
            pltpu.SemaphoreType.DMA,
            pltpu.SemaphoreType.DMA,
        ],
    )(a_s, a_o, s3, o3)


def _premul_kernel(ov_ref, w1a_ref, as_ref, ao_ref):
    din = ov_ref.shape[1]
    w = w1a_ref[...]
    as_ref[...] = _dot(ov_ref[...], w[:din, :])
    ao_ref[...] = _dot(ov_ref[...], w[2 * din:, :])


def _edge_kernel(srow_ref, orow_ref, gsum_ref, pred_ref,
                 w1p_ref, b1a_ref, w1b_ref, b1b_ref,
                 newp_ref, pooled_ref, *, hid, din):
    n = pooled_ref.shape[0]
    eblk = pred_ref.shape[0]

    q = _dot(pred_ref[...], w1p_ref[...])
    t1 = _relu(gsum_ref[...] + q + b1a_ref[...])
    u = _relu(_dot(t1, w1b_ref[...]) + b1b_ref[...])

    newp_ref[...] = u[:, hid:hid + din]

    ii_s = lax.broadcasted_iota(jnp.int32, (n, eblk), 0)
    oh_ss = (ii_s == srow_ref[0]).astype(_F32)
    oh_so = (ii_s == orow_ref[0]).astype(_F32)

    @pl.when(pl.program_id(0) == 0)
    def _():
        pooled_ref[...] = jnp.zeros_like(pooled_ref)

    pooled_ref[...] += (_dot(oh_ss, u[:, :hid])
                        + _dot(oh_so, u[:, hid + din:]))


def _node_kernel(pooled_ref, cnt_ref, w2a_ref, b2a_ref, w2b_ref, b2b_ref,
                 out_ref):
    pm = pooled_ref[...] / jnp.clip(cnt_ref[...], 1.0, None)
    h = _relu(_dot(pm, w2a_ref[...]) + b2a_ref[...])
    out_ref[...] = _relu(_dot(h, w2b_ref[...]) + b2b_ref[...])


def _gtc_layer(obj_vecs, pred_vecs, idx, counts, layer, eblk):
    n, din = obj_vecs.shape
    e = pred_vecs.shape[0]
    s_row3, o_row3, s3, o3 = idx
    (w1a, b1a), (w1b, b1b) = layer['net1']
    (w2a, b2a), (w2b, b2b) = layer['net2']
    hid = w1a.shape[1]
    grid = e // eblk

    a_s, a_o = pl.pallas_call(
        _premul_kernel,
        out_shape=(jax.ShapeDtypeStruct((n, hid), _F32),
                   jax.ShapeDtypeStruct((n, hid), _F32)),
    )(obj_vecs, w1a)

    gsum = _sc_gather_sum(a_s, a_o, s3, o3)

    w1p = w1a[din:2 * din, :]

    new_p, pooled = pl.pallas_call(
        functools.partial(_edge_kernel, hid=hid, din=din),
        grid=(grid,),
        in_specs=[
            pl.BlockSpec((1, 1, eblk), lambda i: (i, 0, 0)),
            pl.BlockSpec((1, 1, eblk), lambda i: (i, 0, 0)),
            pl.BlockSpec((eblk, hid), lambda i: (i, 0)),
            pl.BlockSpec((eblk, din), lambda i: (i, 0)),
            pl.BlockSpec((din, hid), lambda i: (0, 0)),
            pl.BlockSpec((1, hid), lambda i: (0, 0)),
            pl.BlockSpec((hid, 2 * hid + din), lambda i: (0, 0)),
            pl.BlockSpec((1, 2 * hid + din), lambda i: (0, 0)),
        ],
        out_specs=(pl.BlockSpec((eblk, din), lambda i: (i, 0)),
                   pl.BlockSpec((n, hid), lambda i: (0, 0))),
        out_shape=(jax.ShapeDtypeStruct((e, din), _F32),
                   jax.ShapeDtypeStruct((n, hid), _F32)),
    )(s_row3, o_row3, gsum, pred_vecs, w1p,
      b1a.reshape(1, -1), w1b, b1b.reshape(1, -1))

    new_obj = pl.pallas_call(
        _node_kernel,
        out_shape=jax.ShapeDtypeStruct((n, din), _F32),
    )(pooled, counts, w2a, b2a.reshape(1, -1), w2b, b2b.reshape(1, -1))

    return new_obj, new_p


def _heads_kernel(ovb_ref, ovs_ref,
                  wbh0_ref, bbh0_ref, wbh1_ref, bbh1_ref,
                  wbm_ref, bbm_ref, wbv_ref, bbv_ref,
                  wsh0_ref, bsh0_ref, wsh1_ref, bsh1_ref,
                  wsm_ref, bsm_ref, wsv_ref, bsv_ref,
                  mub_ref, lvb_ref, mus_ref, lvs_ref):
    hb = _relu(_dot(ovb_ref[...], wbh0_ref[...]) + bbh0_ref[...])
    hb = _relu(_dot(hb, wbh1_ref[...]) + bbh1_ref[...])
    mub_ref[...] = _dot(hb, wbm_ref[...]) + bbm_ref[...]
    lvb_ref[...] = _dot(hb, wbv_ref[...]) + bbv_ref[...]
    hs = _relu(_dot(ovs_ref[...], wsh0_ref[...]) + bsh0_ref[...])
    hs = _relu(_dot(hs, wsh1_ref[...]) + bsh1_ref[...])
    mus_ref[...] = _dot(hs, wsm_ref[...]) + bsm_ref[...]
    lvs_ref[...] = _dot(hs, wsv_ref[...]) + bsv_ref[...]


def _heads(ovb, ovs, params):
    n = ovb.shape[0]
    (wbh0, bbh0), (wbh1, bbh1) = params['box_mean_var']
    (wbm, bbm), = params['box_mean']
    (wbv, bbv), = params['box_var']
    (wsh0, bsh0), (wsh1, bsh1) = params['shape_mean_var']
    (wsm, bsm), = params['shape_mean']
    (wsv, bsv), = params['shape_var']
    emb = wbm.shape[1]
    out_sh = jax.ShapeDtypeStruct((n, emb), _F32)
    return pl.pallas_call(
        _heads_kernel,
        out_shape=(out_sh, out_sh, out_sh, out_sh),
    )(ovb, ovs,
      wbh0, bbh0.reshape(1, -1), wbh1, bbh1.reshape(1, -1),
      wbm, bbm.reshape(1, -1), wbv, bbv.reshape(1, -1),
      wsh0, bsh0.reshape(1, -1), wsh1, bsh1.reshape(1, -1),
      wsm, bsm.reshape(1, -1), wsv, bsv.reshape(1, -1))


def kernel(objs, triples, boxes_gt, shapes_gt, params):
    e = triples.shape[0]
    n = objs.shape[0]
    eblk = min(1024, e)
    grid = e // eblk

    s = triples[:, 0].astype(jnp.int32)
    p = triples[:, 1].astype(jnp.int32)
    o = triples[:, 2].astype(jnp.int32)
    per_w = e // _NW
    nch = per_w // _CH
    idx = (s.reshape(grid, 1, eblk), o.reshape(grid, 1, eblk),
           s.reshape(_NW, nch, _CH), o.reshape(_NW, nch, _CH))

    ovb, ovs = _node_setup(objs, boxes_gt, shapes_gt, params)
    pvb, pvs = _pred_setup(p, params, eblk)
    counts = _edge_counts(idx[0], idx[1], n, eblk)

    for lb, ls in zip(params['gconv_box'], params['gconv_shape']):
        ovb, pvb = _gtc_layer(ovb, pvb, idx, counts, lb, eblk)
        ovs, pvs = _gtc_layer(ovs, pvs, idx, counts, ls, eblk)

    ov = jnp.concatenate([ovb, ovs], axis=1)
    pv = jnp.concatenate([pvb, pvs], axis=1)
    for layer in params['gconv_shared']:
        ov, pv = _gtc_layer(ov, pv, idx, counts, layer, eblk)

    d = ov.shape[1] // 2
    return _heads(ov[:, :d], ov[:, d:], params)


# speedup vs baseline: 2.5893x; 1.0082x over previous
import functools

import jax
import jax.numpy as jnp
from jax import lax
from jax.experimental import pallas as pl
from jax.experimental.pallas import tpu as pltpu
from jax.experimental.pallas import tpu_sc as plsc


_F32 = jnp.float32
_NC = 2
_NS = 16
_NW = _NC * _NS
_CH = 32


def _dot(a, b):
    return lax.dot_general(a, b, (((1,), (0,)), ((), ())),
                           preferred_element_type=_F32)


def _relu(x):
    return jnp.maximum(x, 0.0)


def _setup_kernel(objs_ref, boxes_ref, shapes_ref, teb_ref, tes_ref,
                  wb_ref, bb_ref, ws_ref, bs_ref, ovb_ref, ovs_ref):
    n = objs_ref.shape[0]
    nobj = teb_ref.shape[0]
    onehot = (lax.broadcasted_iota(jnp.int32, (n, nobj), 1)
              == objs_ref[...]).astype(_F32)
    emb_b = _dot(onehot, teb_ref[...])
    emb_s = _dot(onehot, tes_ref[...])
    bx = _dot(boxes_ref[...], wb_ref[...]) + bb_ref[...]
    sh = _dot(shapes_ref[...], ws_ref[...]) + bs_ref[...]
    ovb_ref[...] = jnp.concatenate([emb_b, bx], axis=1)
    ovs_ref[...] = jnp.concatenate([emb_s, sh], axis=1)


def _node_setup(objs, boxes_gt, shapes_gt, params):
    n = objs.shape[0]
    emb = params['obj_emb_box'].shape[1]
    (wb, bb), = params['box_emb']
    (ws, bs), = params['shape_emb']
    out_sh = jax.ShapeDtypeStruct((n, 2 * emb), _F32)
    return pl.pallas_call(
        _setup_kernel,
        out_shape=(out_sh, out_sh),
    )(objs.reshape(n, 1).astype(jnp.int32), boxes_gt, shapes_gt,
      params['obj_emb_box'], params['obj_emb_shape'],
      wb, bb.reshape(1, -1), ws, bs.reshape(1, -1))


def _pred_kernel(p_ref, tb_ref, ts_ref, pvb_ref, pvs_ref):
    eblk = p_ref.shape[0]
    npred = tb_ref.shape[0]
    onehot = (lax.broadcasted_iota(jnp.int32, (eblk, npred), 1)
              == p_ref[...]).astype(_F32)
    pvb_ref[...] = _dot(onehot, tb_ref[...])
    pvs_ref[...] = _dot(onehot, ts_ref[...])


def _pred_setup(p, params, eblk):
    e = p.shape[0]
    tb = params['pred_emb_box']
    ts = params['pred_emb_shape']
    d = tb.shape[1]
    grid = e // eblk
    out_sh = jax.ShapeDtypeStruct((e, d), _F32)
    return pl.pallas_call(
        _pred_kernel,
        grid=(grid,),
        in_specs=[
            pl.BlockSpec((eblk, 1), lambda i: (i, 0)),
            pl.BlockSpec(tb.shape, lambda i: (0, 0)),
            pl.BlockSpec(ts.shape, lambda i: (0, 0)),
        ],
        out_specs=(pl.BlockSpec((eblk, d), lambda i: (i, 0)),
                   pl.BlockSpec((eblk, d), lambda i: (i, 0))),
        out_shape=(out_sh, out_sh),
    )(p.reshape(e, 1).astype(jnp.int32), tb, ts)


def _counts_kernel(srow_ref, orow_ref, cnt_ref):
    n = cnt_ref.shape[0]
    eblk = srow_ref.shape[-1]

    @pl.when(pl.program_id(0) == 0)
    def _():
        cnt_ref[...] = jnp.zeros_like(cnt_ref)

    ii = lax.broadcasted_iota(jnp.int32, (n, eblk), 0)
    ohs = (ii == srow_ref[0]).astype(_F32)
    oho = (ii == orow_ref[0]).astype(_F32)
    cnt_ref[...] += (jnp.sum(ohs, axis=1, keepdims=True)
                     + jnp.sum(oho, axis=1, keepdims=True))


def _edge_counts(s_row3, o_row3, n, eblk):
    grid = s_row3.shape[0]
    return pl.pallas_call(
        _counts_kernel,
        grid=(grid,),
        in_specs=[
            pl.BlockSpec((1, 1, eblk), lambda i: (i, 0, 0)),
            pl.BlockSpec((1, 1, eblk), lambda i: (i, 0, 0)),
        ],
        out_specs=pl.BlockSpec((n, 1), lambda i: (0, 0)),
        out_shape=jax.ShapeDtypeStruct((n, 1), _F32),
    )(s_row3, o_row3)


def _sc_gather_body(as_hbm, ao_hbm, s3_hbm, o3_hbm, out_hbm,
                    idxs, idxo, bs0, bo0, bs1, bo1, sem0, sem1,
                    *, per_w, nch, hid):
    wid = lax.axis_index("s") * _NC + lax.axis_index("c")
    base_w = wid * per_w
    nv = hid // 16

    pltpu.sync_copy(s3_hbm.at[wid], idxs)
    pltpu.sync_copy(o3_hbm.at[wid], idxo)

    def issue(k, bs, bo, sem):
        @pl.when(k < nch)
        def _():
            pltpu.async_copy(as_hbm.at[idxs.at[k]], bs, sem)
            pltpu.async_copy(ao_hbm.at[idxo.at[k]], bo, sem)

    def drain_add_write(k, bs, bo, sem):
        pltpu.make_async_copy(as_hbm.at[idxs.at[k]], bs, sem).wait()
        pltpu.make_async_copy(ao_hbm.at[idxo.at[k]], bo, sem).wait()

        def row(r, _):
            for c in range(nv):
                sl = pl.ds(c * 16, 16)
                bs[r, sl] = bs[r, sl] + bo[r, sl]
            return ()

        lax.fori_loop(0, _CH, row, ())
        pltpu.sync_copy(bs, out_hbm.at[pl.ds(base_w + k * _CH, _CH)])

    issue(0, bs0, bo0, sem0)

    def pair(i, _):
        k0 = i * 2
        issue(k0 + 1, bs1, bo1, sem1)
        drain_add_write(k0, bs0, bo0, sem0)
        issue(k0 + 2, bs0, bo0, sem0)
        drain_add_write(k0 + 1, bs1, bo1, sem1)
        return ()

    lax.fori_loop(0, nch // 2, pair, ())


def _sc_gather_sum(a_s, a_o, s3, o3):
    n, hid = a_s.shape
    e = s3.shape[0] * s3.shape[1] * s3.shape[2]
    per_w = e // _NW
    nch = per_w // _CH
    mesh = plsc.VectorSubcoreMesh(core_axis_name="c", subcore_axis_name="s")
    body = functools.partial(_sc_gather_body, per_w=per_w, nch=nch, hid=hid)
    buf = pltpu.VMEM((_CH, hid), _F32)
    return pl.kernel(
        body,
        out_type=jax.ShapeDtypeStruct((e, hid), _F32),
        mesh=mesh,
        scratch_types=[
            pltpu.VMEM((nch, _CH), jnp.int32),
            pltpu.VMEM((nch, _CH), jnp.int32),
            buf, buf, buf, buf,
            pltpu.SemaphoreType.DMA,
            pltpu.SemaphoreType.DMA,
        ],
    )(a_s, a_o, s3, o3)


def _premul_kernel(ov_ref, w1a_ref, as_ref, ao_ref):
    din = ov_ref.shape[1]
    w = w1a_ref[...]
    as_ref[...] = _dot(ov_ref[...], w[:din, :])
    ao_ref[...] = _dot(ov_ref[...], w[2 * din:, :])


def _edge_kernel(srow_ref, orow_ref, gsum_ref, pred_ref,
                 w1p_ref, b1a_ref, w1b_ref, b1b_ref,
                 newp_ref, pooled_ref, *, hid, din):
    n = pooled_ref.shape[0]
    eblk = pred_ref.shape[0]

    q = _dot(pred_ref[...], w1p_ref[...])
    t1 = _relu(gsum_ref[...] + q + b1a_ref[...])
    u = _relu(_dot(t1, w1b_ref[...]) + b1b_ref[...])

    newp_ref[...] = u[:, hid:hid + din]

    ii_s = lax.broadcasted_iota(jnp.int32, (n, eblk), 0)
    oh_ss = (ii_s == srow_ref[0]).astype(_F32)
    oh_so = (ii_s == orow_ref[0]).astype(_F32)

    @pl.when(pl.program_id(0) == 0)
    def _():
        pooled_ref[...] = jnp.zeros_like(pooled_ref)

    pooled_ref[...] += (_dot(oh_ss, u[:, :hid])
                        + _dot(oh_so, u[:, hid + din:]))


def _node_kernel(pooled_ref, cnt_ref, w2a_ref, b2a_ref, w2b_ref, b2b_ref,
                 out_ref):
    pm = pooled_ref[...] / jnp.clip(cnt_ref[...], 1.0, None)
    h = _relu(_dot(pm, w2a_ref[...]) + b2a_ref[...])
    out_ref[...] = _relu(_dot(h, w2b_ref[...]) + b2b_ref[...])


def _node_premul_kernel(pooled_ref, cnt_ref, w2a_ref, b2a_ref, w2b_ref,
                        b2b_ref, w1n_ref, out_ref, as_ref, ao_ref):
    pm = pooled_ref[...] / jnp.clip(cnt_ref[...], 1.0, None)
    h = _relu(_dot(pm, w2a_ref[...]) + b2a_ref[...])
    nv = _relu(_dot(h, w2b_ref[...]) + b2b_ref[...])
    out_ref[...] = nv
    dnext = nv.shape[1]
    w = w1n_ref[...]
    as_ref[...] = _dot(nv, w[:dnext, :])
    ao_ref[...] = _dot(nv, w[2 * dnext:, :])


def _gtc_layer(obj_vecs, pred_vecs, idx, counts, layer, eblk,
               pre=None, w1a_next=None):
    n, din = obj_vecs.shape
    e = pred_vecs.shape[0]
    s_row3, o_row3, s3, o3 = idx
    (w1a, b1a), (w1b, b1b) = layer['net1']
    (w2a, b2a), (w2b, b2b) = layer['net2']
    hid = w1a.shape[1]
    grid = e // eblk

    if pre is None:
        a_s, a_o = pl.pallas_call(
            _premul_kernel,
            out_shape=(jax.ShapeDtypeStruct((n, hid), _F32),
                       jax.ShapeDtypeStruct((n, hid), _F32)),
        )(obj_vecs, w1a)
    else:
        a_s, a_o = pre

    gsum = _sc_gather_sum(a_s, a_o, s3, o3)

    w1p = w1a[din:2 * din, :]

    new_p, pooled = pl.pallas_call(
        functools.partial(_edge_kernel, hid=hid, din=din),
        grid=(grid,),
        in_specs=[
            pl.BlockSpec((1, 1, eblk), lambda i: (i, 0, 0)),
            pl.BlockSpec((1, 1, eblk), lambda i: (i, 0, 0)),
            pl.BlockSpec((eblk, hid), lambda i: (i, 0)),
            pl.BlockSpec((eblk, din), lambda i: (i, 0)),
            pl.BlockSpec((din, hid), lambda i: (0, 0)),
            pl.BlockSpec((1, hid), lambda i: (0, 0)),
            pl.BlockSpec((hid, 2 * hid + din), lambda i: (0, 0)),
            pl.BlockSpec((1, 2 * hid + din), lambda i: (0, 0)),
        ],
        out_specs=(pl.BlockSpec((eblk, din), lambda i: (i, 0)),
                   pl.BlockSpec((n, hid), lambda i: (0, 0))),
        out_shape=(jax.ShapeDtypeStruct((e, din), _F32),
                   jax.ShapeDtypeStruct((n, hid), _F32)),
    )(s_row3, o_row3, gsum, pred_vecs, w1p,
      b1a.reshape(1, -1), w1b, b1b.reshape(1, -1))

    if w1a_next is None:
        new_obj = pl.pallas_call(
            _node_kernel,
            out_shape=jax.ShapeDtypeStruct((n, din), _F32),
        )(pooled, counts, w2a, b2a.reshape(1, -1), w2b, b2b.reshape(1, -1))
        return new_obj, new_p, None

    hnext = w1a_next.shape[1]
    new_obj, a_sn, a_on = pl.pallas_call(
        _node_premul_kernel,
        out_shape=(jax.ShapeDtypeStruct((n, din), _F32),
                   jax.ShapeDtypeStruct((n, hnext), _F32),
                   jax.ShapeDtypeStruct((n, hnext), _F32)),
    )(pooled, counts, w2a, b2a.reshape(1, -1), w2b, b2b.reshape(1, -1),
      w1a_next)
    return new_obj, new_p, (a_sn, a_on)


def _heads_kernel(ovb_ref, ovs_ref,
                  wbh0_ref, bbh0_ref, wbh1_ref, bbh1_ref,
                  wbm_ref, bbm_ref, wbv_ref, bbv_ref,
                  wsh0_ref, bsh0_ref, wsh1_ref, bsh1_ref,
                  wsm_ref, bsm_ref, wsv_ref, bsv_ref,
                  mub_ref, lvb_ref, mus_ref, lvs_ref):
    hb = _relu(_dot(ovb_ref[...], wbh0_ref[...]) + bbh0_ref[...])
    hb = _relu(_dot(hb, wbh1_ref[...]) + bbh1_ref[...])
    mub_ref[...] = _dot(hb, wbm_ref[...]) + bbm_ref[...]
    lvb_ref[...] = _dot(hb, wbv_ref[...]) + bbv_ref[...]
    hs = _relu(_dot(ovs_ref[...], wsh0_ref[...]) + bsh0_ref[...])
    hs = _relu(_dot(hs, wsh1_ref[...]) + bsh1_ref[...])
    mus_ref[...] = _dot(hs, wsm_ref[...]) + bsm_ref[...]
    lvs_ref[...] = _dot(hs, wsv_ref[...]) + bsv_ref[...]


def _heads(ovb, ovs, params):
    n = ovb.shape[0]
    (wbh0, bbh0), (wbh1, bbh1) = params['box_mean_var']
    (wbm, bbm), = params['box_mean']
    (wbv, bbv), = params['box_var']
    (wsh0, bsh0), (wsh1, bsh1) = params['shape_mean_var']
    (wsm, bsm), = params['shape_mean']
    (wsv, bsv), = params['shape_var']
    emb = wbm.shape[1]
    out_sh = jax.ShapeDtypeStruct((n, emb), _F32)
    return pl.pallas_call(
        _heads_kernel,
        out_shape=(out_sh, out_sh, out_sh, out_sh),
    )(ovb, ovs,
      wbh0, bbh0.reshape(1, -1), wbh1, bbh1.reshape(1, -1),
      wbm, bbm.reshape(1, -1), wbv, bbv.reshape(1, -1),
      wsh0, bsh0.reshape(1, -1), wsh1, bsh1.reshape(1, -1),
      wsm, bsm.reshape(1, -1), wsv, bsv.reshape(1, -1))


def kernel(objs, triples, boxes_gt, shapes_gt, params):
    e = triples.shape[0]
    n = objs.shape[0]
    eblk = min(1024, e)
    grid = e // eblk

    s = triples[:, 0].astype(jnp.int32)
    p = triples[:, 1].astype(jnp.int32)
    o = triples[:, 2].astype(jnp.int32)
    per_w = e // _NW
    nch = per_w // _CH
    idx = (s.reshape(grid, 1, eblk), o.reshape(grid, 1, eblk),
           s.reshape(_NW, nch, _CH), o.reshape(_NW, nch, _CH))

    ovb, ovs = _node_setup(objs, boxes_gt, shapes_gt, params)
    pvb, pvs = _pred_setup(p, params, eblk)
    counts = _edge_counts(idx[0], idx[1], n, eblk)

    lbs, lss = params['gconv_box'], params['gconv_shape']
    pre_b = pre_s = None
    for i, (lb, ls) in enumerate(zip(lbs, lss)):
        nb = lbs[i + 1]['net1'][0][0] if i + 1 < len(lbs) else None
        ns_ = lss[i + 1]['net1'][0][0] if i + 1 < len(lss) else None
        ovb, pvb, pre_b = _gtc_layer(ovb, pvb, idx, counts, lb, eblk,
                                     pre_b, nb)
        ovs, pvs, pre_s = _gtc_layer(ovs, pvs, idx, counts, ls, eblk,
                                     pre_s, ns_)

    ov = jnp.concatenate([ovb, ovs], axis=1)
    pv = jnp.concatenate([pvb, pvs], axis=1)
    lsh = params['gconv_shared']
    pre = None
    for i, layer in enumerate(lsh):
        nx = lsh[i + 1]['net1'][0][0] if i + 1 < len(lsh) else None
        ov, pv, pre = _gtc_layer(ov, pv, idx, counts, layer, eblk, pre, nx)

    d = ov.shape[1] // 2
    return _heads(ov[:, :d], ov[:, d:], params)
